# baseline (speedup 1.0000x reference)
import functools

import jax
import jax.numpy as jnp
from jax import lax
from jax.experimental import pallas as pl
from jax.experimental.pallas import tpu as pltpu
from jax.experimental.pallas import tpu_sc as plsc

VOCAB = 1000000
D = 32
NC = 2
NS = 16
NW = NC * NS
TOTAL = 16384 * 50
ROWS_PER_W = TOTAL // NW
CHUNK = 128
N_CHUNKS = ROWS_PER_W // CHUNK
K = 5
NBUF = 4
GROUP = K * CHUNK
N_GROUPS = N_CHUNKS // K
N_ITERS = N_GROUPS // NBUF


TW = 512


def _tc_transpose(table_t):
    def body(in_ref, out_ref):
        blk = in_ref[...]
        t = jnp.transpose(blk, (1, 0))
        t3 = t.reshape(TW // 4, 4, D)
        out_ref[...] = jnp.concatenate([t3[:, a, :] for a in range(4)], axis=1)

    return pl.pallas_call(
        body,
        grid=(pl.cdiv(VOCAB, TW),),
        in_specs=[pl.BlockSpec((D, TW), lambda i: (0, i))],
        out_specs=pl.BlockSpec((TW // 4, 4 * D), lambda i: (i, 0)),
        out_shape=jax.ShapeDtypeStruct((VOCAB // 4, 4 * D), jnp.float32),
    )(table_t)


def _build_gather():
    mesh = plsc.VectorSubcoreMesh(core_axis_name="c", subcore_axis_name="s")

    @functools.partial(
        pl.kernel,
        mesh=mesh,
        compiler_params=pltpu.CompilerParams(use_tc_tiling_on_sc=False),
        out_type=jax.ShapeDtypeStruct((TOTAL, D), jnp.float32),
        scratch_types=[
            pltpu.VMEM((N_CHUNKS, CHUNK), jnp.int32),
            pltpu.VMEM((NBUF, GROUP, D), jnp.float32),
        ]
        + [pltpu.SemaphoreType.DMA] * (2 * NBUF),
    )
    def gather_kernel(table_hbm, idx_hbm, out_hbm, idx_v, rows_v, *sems):
        gsem = sems[:NBUF]
        wsem = sems[NBUF:]
        wid = lax.axis_index("s") * NC + lax.axis_index("c")
        base = wid * ROWS_PER_W
        pltpu.sync_copy(idx_hbm.at[wid], idx_v)

        def fire_gathers(g, b):
            for j in range(K):
                pltpu.async_copy(
                    table_hbm.at[idx_v.at[g * K + j]],
                    rows_v.at[b, pl.ds(j * CHUNK, CHUNK)],
                    gsem[b],
                )

        def drain_gathers(b):
            pltpu.make_async_copy(
                table_hbm.at[pl.ds(0, GROUP)], rows_v.at[b], gsem[b]
            ).wait()

        def out_slice(g):
            return out_hbm.at[pl.ds(base + g * GROUP, GROUP)]

        def fire_write(g, b):
            pltpu.async_copy(rows_v.at[b], out_slice(g), wsem[b])

        def wait_write(g, b):
            pltpu.make_async_copy(rows_v.at[b], out_slice(g), wsem[b]).wait()

        for b in range(NBUF):
            fire_gathers(b, b)

        def body(i, carry):
            g0 = i * NBUF
            for b in range(NBUF):
                drain_gathers(b)
                fire_write(g0 + b, b)
            for b in range(NBUF):
                wait_write(g0 + b, b)
                fire_gathers(g0 + NBUF + b, b)
            return carry

        lax.fori_loop(0, N_ITERS - 1, body, 0, unroll=False)

        g0 = (N_ITERS - 1) * NBUF
        for b in range(NBUF):
            drain_gathers(b)
            fire_write(g0 + b, b)
        for b in range(NBUF):
            wait_write(g0 + b, b)

    return gather_kernel


_gather = _build_gather()


def kernel(x, table):
    flat = _tc_transpose(table.T).reshape(VOCAB, D)
    idx = x.T.astype(jnp.int32).reshape(NW, N_CHUNKS, CHUNK)
    out = _gather(flat, idx)
    return jnp.transpose(out.reshape(x.shape[1], x.shape[0], D), (1, 0, 2))


# speedup vs baseline: 1.7356x; 1.7356x over previous
import functools

import jax
import jax.numpy as jnp
from jax import lax
from jax.experimental import pallas as pl
from jax.experimental.pallas import tpu as pltpu
from jax.experimental.pallas import tpu_sc as plsc

VOCAB = 1000000
D = 32
NC = 2
NS = 16
NW = NC * NS
TOTAL = 16384 * 50
ROWS_PER_W = TOTAL // NW
CHUNK = 128
N_CHUNKS = ROWS_PER_W // CHUNK
K = 5
NBUF = 4
GROUP = K * CHUNK
N_GROUPS = N_CHUNKS // K
N_ITERS = N_GROUPS // NBUF


def _build_gather():
    mesh = plsc.VectorSubcoreMesh(core_axis_name="c", subcore_axis_name="s")

    @functools.partial(
        pl.kernel,
        mesh=mesh,
        compiler_params=pltpu.CompilerParams(use_tc_tiling_on_sc=False),
        out_type=jax.ShapeDtypeStruct((TOTAL, D), jnp.float32),
        scratch_types=[
            pltpu.VMEM((N_CHUNKS, CHUNK), jnp.int32),
            pltpu.VMEM((NBUF, GROUP, D), jnp.float32),
        ]
        + [pltpu.SemaphoreType.DMA] * (2 * NBUF),
    )
    def gather_kernel(table_hbm, idx_hbm, out_hbm, idx_v, rows_v, *sems):
        gsem = sems[:NBUF]
        wsem = sems[NBUF:]
        wid = lax.axis_index("s") * NC + lax.axis_index("c")
        base = wid * ROWS_PER_W
        pltpu.sync_copy(idx_hbm.at[wid], idx_v)

        def fire_gathers(g, b):
            for j in range(K):
                pltpu.async_copy(
                    table_hbm.at[idx_v.at[g * K + j]],
                    rows_v.at[b, pl.ds(j * CHUNK, CHUNK)],
                    gsem[b],
                )

        def drain_gathers(b):
            pltpu.make_async_copy(
                table_hbm.at[pl.ds(0, GROUP)], rows_v.at[b], gsem[b]
            ).wait()

        def out_slice(g):
            return out_hbm.at[pl.ds(base + g * GROUP, GROUP)]

        def fire_write(g, b):
            pltpu.async_copy(rows_v.at[b], out_slice(g), wsem[b])

        def wait_write(g, b):
            pltpu.make_async_copy(rows_v.at[b], out_slice(g), wsem[b]).wait()

        for b in range(NBUF):
            fire_gathers(b, b)

        def body(i, carry):
            g0 = i * NBUF
            for b in range(NBUF):
                drain_gathers(b)
                fire_write(g0 + b, b)
            for b in range(NBUF):
                wait_write(g0 + b, b)
                fire_gathers(g0 + NBUF + b, b)
            return carry

        lax.fori_loop(0, N_ITERS - 1, body, 0, unroll=False)

        g0 = (N_ITERS - 1) * NBUF
        for b in range(NBUF):
            drain_gathers(b)
            fire_write(g0 + b, b)
        for b in range(NBUF):
            wait_write(g0 + b, b)

    return gather_kernel


_gather = _build_gather()


def kernel(x, table):
    idx = x.T.astype(jnp.int32).reshape(NW, N_CHUNKS, CHUNK)
    out = _gather(table, idx)
    return jnp.transpose(out.reshape(x.shape[1], x.shape[0], D), (1, 0, 2))
